# Initial kernel scaffold; baseline (speedup 1.0000x reference)
import functools

import jax
import jax.numpy as jnp
from jax import lax
from jax.experimental import pallas as pl
from jax.experimental.pallas import tpu as pltpu
from jax.experimental.pallas import tpu_sc as plsc

N_GRID = 128
SIDE = 1.5
NPL = N_GRID + 1
NPL2 = NPL * NPL
W_FEAT = 16
B = 524288
C = 512
L = 16
OUT_W = 1 + 3 + W_FEAT

_OFF = [(di, dj, dk) for di in (0, 1) for dj in (0, 1) for dk in (0, 1)]


def _build():
  info = plsc.get_sparse_core_info()
  NC, NS = info.num_cores, info.num_subcores
  NW = NC * NS
  PW = B // NW
  NCHUNK = PW // C
  NIDX = 8 * C
  G = NIDX // 128

  mesh = plsc.VectorSubcoreMesh(core_axis_name="c", subcore_axis_name="s")

  @functools.partial(
      pl.kernel,
      mesh=mesh,
      out_type=jax.ShapeDtypeStruct((B, OUT_W), jnp.float32),
      scratch_types=[
          pltpu.VMEM((3, C), jnp.float32),
          pltpu.VMEM((G, 128), jnp.int32),
          pltpu.VMEM((8, C), jnp.float32),
          pltpu.VMEM((NIDX, W_FEAT), jnp.float32),
          pltpu.VMEM((NIDX,), jnp.float32),
          pltpu.VMEM((C, OUT_W), jnp.float32),
          pltpu.SemaphoreType.DMA,
          pltpu.SemaphoreType.DMA,
      ],
  )
  def grid_embed(xyzt_hbm, val_hbm, feat_hbm, out_hbm,
                 xyz_v, idx_v, w_v, frows_v, vrows_v, out_v, semf, semv):
    wid = lax.axis_index("s") * NC + lax.axis_index("c")
    iota = lax.iota(jnp.int32, L)
    zeros_i = jnp.zeros((L,), jnp.int32)

    def chunk_body(t, carry):
      base = wid * PW + t * C

      for d in range(3):
        pltpu.sync_copy(xyzt_hbm.at[d, pl.ds(base, C)], xyz_v.at[d])

      for i in range(C // L):
        ix, fr = [], []
        vmask = None
        for d in range(3):
          xd = xyz_v[d, pl.ds(i * L, L)]
          ok = (xd >= -0.75) & (xd <= 0.75)
          vmask = ok if vmask is None else (vmask & ok)
          u = (xd + 0.75) / SIDE * float(N_GRID)
          u = jnp.minimum(jnp.maximum(u, 0.0), float(N_GRID))
          ii = jnp.minimum(u.astype(jnp.int32), N_GRID - 1)
          ix.append(ii)
          fr.append(u - ii.astype(jnp.float32))
        validf = jnp.where(vmask, 1.0, 0.0).astype(jnp.float32)
        base_idx = ix[0] * NPL2 + ix[1] * NPL + ix[2]

        rows = i * L + iota
        for d in range(3):
          plsc.store_scatter(out_v, [rows, jnp.full((L,), d + 1, jnp.int32)],
                             xyz_v[d, pl.ds(i * L, L)])

        for c, (di, dj, dk) in enumerate(_OFF):
          idx_c = base_idx + (di * NPL2 + dj * NPL + dk)
          wc = ((fr[0] if di else 1.0 - fr[0])
                * (fr[1] if dj else 1.0 - fr[1])
                * (fr[2] if dk else 1.0 - fr[2]) * validf)
          flat = c * C + i * L
          idx_v[flat // 128, pl.ds(flat % 128, L)] = idx_c
          w_v[c, pl.ds(i * L, L)] = wc

      handles = []
      for g in range(G):
        handles.append(pltpu.async_copy(
            feat_hbm.at[idx_v.at[g]], frows_v.at[pl.ds(g * 128, 128)], semf))
        handles.append(pltpu.async_copy(
            val_hbm.at[idx_v.at[g]], vrows_v.at[pl.ds(g * 128, 128)], semv))
      for h in handles:
        h.wait()

      for i in range(C // L):
        acc = jnp.zeros((L,), jnp.float32)
        for c in range(8):
          acc = acc + w_v[c, pl.ds(i * L, L)] * vrows_v[pl.ds(c * C + i * L, L)]
        plsc.store_scatter(out_v, [i * L + iota, zeros_i], acc)

      def feat_body(p, fcarry):
        acc = jnp.zeros((W_FEAT,), jnp.float32)
        for c in range(8):
          acc = acc + w_v[c, p] * frows_v[c * C + p, :]
        out_v[p, pl.ds(1 + 3, W_FEAT)] = acc
        return fcarry

      lax.fori_loop(0, C, feat_body, 0)

      pltpu.sync_copy(out_v, out_hbm.at[pl.ds(base, C)])
      return carry

    lax.fori_loop(0, NCHUNK, chunk_body, 0)

  return grid_embed


_GRID_EMBED = None


def kernel(xyz, grid_value_param, grid_feat_param):
  global _GRID_EMBED
  if _GRID_EMBED is None:
    _GRID_EMBED = _build()
  xyzt = jnp.transpose(xyz)
  val = jnp.reshape(grid_value_param, (-1,))
  return _GRID_EMBED(xyzt, val, grid_feat_param)


# speedup vs baseline: 2.0799x; 2.0799x over previous
import functools

import jax
import jax.numpy as jnp
from jax import lax
from jax.experimental import pallas as pl
from jax.experimental.pallas import tpu as pltpu
from jax.experimental.pallas import tpu_sc as plsc

N_GRID = 128
SIDE = 1.5
NPL = N_GRID + 1
NPL2 = NPL * NPL
W_FEAT = 16
B = 524288
C = 512
L = 16
OUT_W = 1 + 3 + W_FEAT

_OFF = [(di, dj, dk) for di in (0, 1) for dj in (0, 1) for dk in (0, 1)]


def _build():
  info = plsc.get_sparse_core_info()
  NC, NS = info.num_cores, info.num_subcores
  NW = NC * NS
  PW = B // NW
  NCHUNK = PW // C
  NIDX = 8 * C
  G = NIDX // 128

  mesh = plsc.VectorSubcoreMesh(core_axis_name="c", subcore_axis_name="s")

  @functools.partial(
      pl.kernel,
      mesh=mesh,
      compiler_params=pltpu.CompilerParams(
          needs_layout_passes=False, use_tc_tiling_on_sc=False),
      out_type=jax.ShapeDtypeStruct((B * OUT_W,), jnp.float32),
      scratch_types=[
          pltpu.VMEM((C,), jnp.float32),
          pltpu.VMEM((C,), jnp.float32),
          pltpu.VMEM((C,), jnp.float32),
          pltpu.VMEM((NIDX,), jnp.int32),
          pltpu.VMEM((NIDX,), jnp.float32),
          pltpu.VMEM((NIDX, W_FEAT), jnp.float32),
          pltpu.VMEM((NIDX,), jnp.float32),
          pltpu.VMEM((C * OUT_W,), jnp.float32),
          pltpu.SemaphoreType.DMA,
          pltpu.SemaphoreType.DMA,
      ],
  )
  def grid_embed(x_hbm, y_hbm, z_hbm, val_hbm, feat_hbm, out_hbm,
                 xs_v, ys_v, zs_v, idx_v, w_v, frows_v, vrows_v, out_v,
                 semf, semv):
    wid = lax.axis_index("s") * NC + lax.axis_index("c")
    iota = lax.iota(jnp.int32, L)

    def chunk_body(t, carry):
      base = wid * PW + t * C

      pltpu.sync_copy(x_hbm.at[pl.ds(base, C)], xs_v)
      pltpu.sync_copy(y_hbm.at[pl.ds(base, C)], ys_v)
      pltpu.sync_copy(z_hbm.at[pl.ds(base, C)], zs_v)

      for i in range(C // L):
        orow = i * L * OUT_W + iota * OUT_W
        ix, fr = [], []
        vmask = None
        for d, cref in enumerate((xs_v, ys_v, zs_v)):
          xd = cref[pl.ds(i * L, L)]
          ok = (xd >= -0.75) & (xd <= 0.75)
          vmask = ok if vmask is None else (vmask & ok)
          u = (xd + 0.75) / SIDE * float(N_GRID)
          u = jnp.minimum(jnp.maximum(u, 0.0), float(N_GRID))
          ii = jnp.minimum(u.astype(jnp.int32), N_GRID - 1)
          ix.append(ii)
          fr.append(u - ii.astype(jnp.float32))
          plsc.store_scatter(out_v, [orow + (1 + d)], xd)
        validf = jnp.where(vmask, 1.0, 0.0).astype(jnp.float32)
        base_idx = ix[0] * NPL2 + ix[1] * NPL + ix[2]

        for c, (di, dj, dk) in enumerate(_OFF):
          idx_c = base_idx + (di * NPL2 + dj * NPL + dk)
          wc = ((fr[0] if di else 1.0 - fr[0])
                * (fr[1] if dj else 1.0 - fr[1])
                * (fr[2] if dk else 1.0 - fr[2]) * validf)
          idx_v[pl.ds(c * C + i * L, L)] = idx_c
          w_v[pl.ds(c * C + i * L, L)] = wc

      handles = []
      for g in range(G):
        gsl = pl.ds(g * 128, 128)
        handles.append(pltpu.async_copy(
            feat_hbm.at[idx_v.at[gsl]], frows_v.at[gsl], semf))
        handles.append(pltpu.async_copy(
            val_hbm.at[idx_v.at[gsl]], vrows_v.at[gsl], semv))
      for h in handles:
        h.wait()

      def combine_body(i, fcarry):
        p0 = i * L
        rows = p0 + iota
        orow = p0 * OUT_W + iota * OUT_W
        ws = [w_v[pl.ds(c * C + p0, L)] for c in range(8)]

        vacc = jnp.zeros((L,), jnp.float32)
        for c in range(8):
          vacc = vacc + ws[c] * vrows_v[pl.ds(c * C + p0, L)]
        plsc.store_scatter(out_v, [orow], vacc)

        for d in range(W_FEAT):
          col = jnp.full((L,), d, jnp.int32)
          facc = jnp.zeros((L,), jnp.float32)
          for c in range(8):
            facc = facc + ws[c] * plsc.load_gather(
                frows_v, [c * C + rows, col])
          plsc.store_scatter(out_v, [orow + (4 + d)], facc)
        return fcarry

      lax.fori_loop(0, C // L, combine_body, 0)

      pltpu.sync_copy(out_v, out_hbm.at[pl.ds(base * OUT_W, C * OUT_W)])
      return carry

    lax.fori_loop(0, NCHUNK, chunk_body, 0)

  return grid_embed


_GRID_EMBED = None


def kernel(xyz, grid_value_param, grid_feat_param):
  global _GRID_EMBED
  if _GRID_EMBED is None:
    _GRID_EMBED = _build()
  x = xyz[:, 0]
  y = xyz[:, 1]
  z = xyz[:, 2]
  val = jnp.reshape(grid_value_param, (-1,))
  out = _GRID_EMBED(x, y, z, val, grid_feat_param)
  return jnp.reshape(out, (B, OUT_W))
